# baseline (device time: 104883 ns/iter reference)
import jax
import jax.numpy as jnp
from jax import lax
from jax.experimental import pallas as pl
from jax.experimental.pallas import tpu as pltpu

N_DEV = 4
EPS = 1e-5


def kernel(x, Wp):
    B, Hs, W, C = x.shape
    Cout = Wp.shape[1]
    n_total = float(Hs * N_DEV * W)

    CH = 32
    NCH = Hs // CH
    HBLK = 16
    NH = Hs // HBLK

    def body(x_hbm, wp_ref, out_ref,
             xin_ref, xbf_ref, acc_ref, sc_ref, comm_ref,
             in_sems, send_sems, recv_sems):
        step = pl.program_id(0)

        @pl.when(step == 0)
        def _():
            def chunk_copy(i, slot):
                return pltpu.make_async_copy(
                    x_hbm.at[:, pl.ds(i * CH, CH)],
                    xin_ref.at[slot],
                    in_sems.at[slot],
                )

            chunk_copy(0, 0).start()
            for i in range(NCH):
                slot = i % 2
                chunk_copy(i, slot).wait()
                if i + 1 < NCH:
                    chunk_copy(i + 1, (i + 1) % 2).start()
                xb = xin_ref[slot]
                s = jnp.sum(xb, axis=(1, 2))
                ss = jnp.sum(xb * xb, axis=(1, 2))
                if i == 0:
                    acc_ref[0] = s
                    acc_ref[1] = ss
                else:
                    acc_ref[0] += s
                    acc_ref[1] += ss

            my = lax.axis_index("i")

            barrier = pltpu.get_barrier_semaphore()
            for off in (1, 2, 3):
                pl.semaphore_signal(
                    barrier, inc=1,
                    device_id=((my + off) % N_DEV,),
                    device_id_type=pl.DeviceIdType.MESH,
                )
            pl.semaphore_wait(barrier, N_DEV - 1)

            comm_ref[my] = acc_ref[...]

            sends = []
            for off in (1, 2, 3):
                tgt = (my + off) % N_DEV
                rdma = pltpu.make_async_remote_copy(
                    src_ref=comm_ref.at[my],
                    dst_ref=comm_ref.at[my],
                    send_sem=send_sems.at[off - 1],
                    recv_sem=recv_sems.at[my],
                    device_id=(tgt,),
                    device_id_type=pl.DeviceIdType.MESH,
                )
                rdma.start()
                sends.append(rdma)

            for off in (1, 2, 3):
                src = (my - off) % N_DEV
                recv = pltpu.make_async_remote_copy(
                    src_ref=comm_ref.at[src],
                    dst_ref=comm_ref.at[src],
                    send_sem=send_sems.at[off - 1],
                    recv_sem=recv_sems.at[src],
                    device_id=(src,),
                    device_id_type=pl.DeviceIdType.MESH,
                )
                recv.wait_recv()
            for rdma in sends:
                rdma.wait_send()

            tot = comm_ref[0] + comm_ref[1] + comm_ref[2] + comm_ref[3]
            mean = tot[0] * (1.0 / n_total)
            ex2 = tot[1] * (1.0 / n_total)
            var = ex2 - mean * mean
            rstd = lax.rsqrt(var + EPS)
            sc_ref[0] = rstd
            sc_ref[1] = -mean * rstd

        xb = xbf_ref[:, pl.ds(step * HBLK, HBLK)]
        scale = sc_ref[0].astype(jnp.bfloat16)[:, None, None, :]
        shift = sc_ref[1].astype(jnp.bfloat16)[:, None, None, :]
        hn = xb * scale + shift
        a = (0.5 * hn) * (jnp.tanh(0.5 * hn) + 1.0)
        ab = a.reshape(B * HBLK * W, C)
        o = jnp.dot(ab, wp_ref[...].astype(jnp.bfloat16),
                    preferred_element_type=jnp.float32)
        out_ref[...] = o.astype(jnp.bfloat16).reshape(B, HBLK, W, Cout)

    out = pl.pallas_call(
        body,
        grid=(NH,),
        in_specs=[
            pl.BlockSpec(memory_space=pl.ANY),
            pl.BlockSpec((C, Cout), lambda h: (0, 0)),
        ],
        out_specs=pl.BlockSpec((B, HBLK, W, Cout), lambda h: (0, h, 0, 0)),
        out_shape=jax.ShapeDtypeStruct((B, Hs, W, Cout), jnp.bfloat16),
        scratch_shapes=[
            pltpu.VMEM((2, B, CH, W, C), jnp.float32),
            pltpu.VMEM((B, Hs, W, C), jnp.bfloat16),
            pltpu.VMEM((2, B, C), jnp.float32),
            pltpu.VMEM((2, B, C), jnp.float32),
            pltpu.VMEM((N_DEV, 2, B, C), jnp.float32),
            pltpu.SemaphoreType.DMA((2,)),
            pltpu.SemaphoreType.DMA((3,)),
            pltpu.SemaphoreType.DMA((N_DEV,)),
        ],
        compiler_params=pltpu.CompilerParams(
            collective_id=0,
            dimension_semantics=("arbitrary",),
            vmem_limit_bytes=64 * 1024 * 1024,
        ),
    )(x, Wp)
    return out


# device time: 72559 ns/iter; 1.4455x vs baseline; 1.4455x over previous
import jax
import jax.numpy as jnp
from jax import lax
from jax.experimental import pallas as pl
from jax.experimental.pallas import tpu as pltpu

N_DEV = 4
EPS = 1e-5


def kernel(x, Wp):
    B, Hs, W, C = x.shape
    Cout = Wp.shape[1]
    n_total = float(Hs * N_DEV * W)

    HBLK_A = 32
    NH_A = Hs // HBLK_A
    HBLK_B = 32
    NH_B = Hs // HBLK_B

    def stats_body(x_ref, stats_ref, acc_ref, comm_ref, send_sems, recv_sems):
        h = pl.program_id(0)
        xb = x_ref[...]
        s = jnp.sum(xb, axis=(1, 2))
        ss = jnp.sum(xb * xb, axis=(1, 2))

        @pl.when(h == 0)
        def _():
            acc_ref[0] = s
            acc_ref[1] = ss

        @pl.when(h > 0)
        def _():
            acc_ref[0] += s
            acc_ref[1] += ss

        @pl.when(h == NH_A - 1)
        def _():
            my = lax.axis_index("i")

            barrier = pltpu.get_barrier_semaphore()
            for off in (1, 2, 3):
                pl.semaphore_signal(
                    barrier, inc=1,
                    device_id=((my + off) % N_DEV,),
                    device_id_type=pl.DeviceIdType.MESH,
                )
            pl.semaphore_wait(barrier, N_DEV - 1)

            comm_ref[my] = acc_ref[...]

            sends = []
            for off in (1, 2, 3):
                tgt = (my + off) % N_DEV
                rdma = pltpu.make_async_remote_copy(
                    src_ref=comm_ref.at[my],
                    dst_ref=comm_ref.at[my],
                    send_sem=send_sems.at[off - 1],
                    recv_sem=recv_sems.at[my],
                    device_id=(tgt,),
                    device_id_type=pl.DeviceIdType.MESH,
                )
                rdma.start()
                sends.append(rdma)

            for off in (1, 2, 3):
                src = (my - off) % N_DEV
                recv = pltpu.make_async_remote_copy(
                    src_ref=comm_ref.at[src],
                    dst_ref=comm_ref.at[src],
                    send_sem=send_sems.at[off - 1],
                    recv_sem=recv_sems.at[src],
                    device_id=(src,),
                    device_id_type=pl.DeviceIdType.MESH,
                )
                recv.wait_recv()
            for rdma in sends:
                rdma.wait_send()

            tot = comm_ref[0] + comm_ref[1] + comm_ref[2] + comm_ref[3]
            mean = tot[0] * (1.0 / n_total)
            ex2 = tot[1] * (1.0 / n_total)
            var = ex2 - mean * mean
            rstd = lax.rsqrt(var + EPS)
            stats_ref[0] = 0.5 * rstd
            stats_ref[1] = -0.5 * mean * rstd

    stats = pl.pallas_call(
        stats_body,
        grid=(NH_A,),
        in_specs=[
            pl.BlockSpec((B, HBLK_A, W, C), lambda h: (0, h, 0, 0)),
        ],
        out_specs=pl.BlockSpec((2, B, C), lambda h: (0, 0, 0)),
        out_shape=jax.ShapeDtypeStruct((2, B, C), jnp.float32),
        scratch_shapes=[
            pltpu.VMEM((2, B, C), jnp.float32),
            pltpu.VMEM((N_DEV, 2, B, C), jnp.float32),
            pltpu.SemaphoreType.DMA((3,)),
            pltpu.SemaphoreType.DMA((N_DEV,)),
        ],
        compiler_params=pltpu.CompilerParams(
            collective_id=0,
            dimension_semantics=("arbitrary",),
        ),
    )(x)

    def apply_body(x_ref, stats_ref, wp_ref, out_ref):
        xb = x_ref[...].astype(jnp.bfloat16)
        scale = stats_ref[0].astype(jnp.bfloat16)[:, None, None, :]
        shift = stats_ref[1].astype(jnp.bfloat16)[:, None, None, :]
        hh = xb * scale + shift
        a = hh * jnp.tanh(hh) + hh
        ab = a.reshape(B * HBLK_B * W, C)
        wb = wp_ref[...].astype(jnp.bfloat16)
        o = jnp.dot(ab, wb, preferred_element_type=jnp.float32)
        out_ref[...] = o.astype(jnp.bfloat16).reshape(B, HBLK_B, W, Cout)

    out = pl.pallas_call(
        apply_body,
        grid=(NH_B,),
        in_specs=[
            pl.BlockSpec((B, HBLK_B, W, C), lambda h: (0, h, 0, 0)),
            pl.BlockSpec((2, B, C), lambda h: (0, 0, 0)),
            pl.BlockSpec((C, Cout), lambda h: (0, 0)),
        ],
        out_specs=pl.BlockSpec((B, HBLK_B, W, Cout), lambda h: (0, h, 0, 0)),
        out_shape=jax.ShapeDtypeStruct((B, Hs, W, Cout), jnp.bfloat16),
        compiler_params=pltpu.CompilerParams(
            dimension_semantics=("parallel",),
            vmem_limit_bytes=64 * 1024 * 1024,
        ),
    )(x, stats, Wp)
    return out
